# baseline (speedup 1.0000x reference)
import functools

import jax
import jax.numpy as jnp
from jax import lax
from jax.experimental import pallas as pl
from jax.experimental.pallas import tpu as pltpu
from jax.experimental.pallas import tpu_sc as plsc

N_NODES = 10000
D = 128
E = 320000
L = 16
NUM_CORES = 2
NUM_SUBCORES = 16
N_WORKERS = NUM_CORES * NUM_SUBCORES
EPW = E // N_WORKERS
CHUNK = 80
N_CHUNKS = EPW // CHUNK
UNROLL = 4


def _decoder_body(z_hbm, src_hbm, dst_hbm, out_hbm,
                  z_sh, idx_s_v, idx_d_v, rows_s, rows_d, out_v, sem):
    sid = lax.axis_index("s")
    wid = sid * NUM_CORES + lax.axis_index("c")
    base = wid * EPW

    @pl.when(sid == 0)
    def _():
        pltpu.sync_copy(z_hbm, z_sh)

    pltpu.sync_copy(src_hbm.at[pl.ds(base, EPW)], idx_s_v)
    pltpu.sync_copy(dst_hbm.at[pl.ds(base, EPW)], idx_d_v)
    plsc.subcore_barrier()

    lane = lax.iota(jnp.int32, L)
    rot = [jnp.bitwise_and(lane + (1 << r), L - 1) for r in range(4)]

    dnums = lax.GatherDimensionNumbers(
        offset_dims=(), collapsed_slice_dims=(0,), start_index_map=(0,))

    def _perm(v, idx):
        return lax.gather(v, idx[:, None], dnums, (1,),
                          mode=lax.GatherScatterMode.PROMISE_IN_BOUNDS)

    def _sum16(v):
        for r in range(4):
            v = v + _perm(v, rot[r])
        return v

    def _copies(ci, buf):
        off = ci * CHUNK
        return (
            pltpu.make_async_copy(
                z_sh.at[idx_s_v.at[pl.ds(off, CHUNK)]], rows_s.at[buf], sem),
            pltpu.make_async_copy(
                z_sh.at[idx_d_v.at[pl.ds(off, CHUNK)]], rows_d.at[buf], sem),
        )

    for c in _copies(0, 0):
        c.start()

    def chunk_body(ci, carry):
        off = ci * CHUNK
        buf = lax.rem(ci, 2)
        ci_next = jnp.minimum(ci + 1, N_CHUNKS - 1)
        for c in _copies(ci_next, 1 - buf):
            c.start()
        for c in _copies(ci, buf):
            c.wait()

        @plsc.parallel_loop(0, CHUNK // L)
        def group_body(g):
            res = jnp.zeros((L,), jnp.float32)
            for u in range(L):
                e = g * L + u
                acc0 = None
                acc1 = None
                for k in range(D // (2 * L)):
                    sv = rows_s[buf, e, pl.ds(k * 2 * L, 2 * L)]
                    dv = rows_d[buf, e, pl.ds(k * 2 * L, 2 * L)]
                    w = plsc.bitcast(sv * dv, jnp.uint32)
                    p0 = plsc.bitcast(w << jnp.uint32(16), jnp.float32)
                    p1 = plsc.bitcast(w & jnp.uint32(0xFFFF0000), jnp.float32)
                    acc0 = p0 if acc0 is None else acc0 + p0
                    acc1 = p1 if acc1 is None else acc1 + p1
                res = jnp.where(lane == u, _sum16(acc0 + acc1), res)
            out_v[pl.ds(off + g * L, L)] = res

        return carry

    lax.fori_loop(0, N_CHUNKS, chunk_body, 0)

    for c in _copies(N_CHUNKS - 1, N_CHUNKS % 2):
        c.wait()

    @plsc.parallel_loop(0, EPW // L, unroll=4)
    def sig_body(i):
        x = out_v[pl.ds(i * L, L)]
        out_v[pl.ds(i * L, L)] = 1.0 / (1.0 + jnp.exp(-x))

    pltpu.sync_copy(out_v, out_hbm.at[pl.ds(base, EPW)])


_decoder = functools.partial(
    pl.kernel,
    out_type=jax.ShapeDtypeStruct((E,), jnp.float32),
    mesh=plsc.VectorSubcoreMesh(core_axis_name="c", subcore_axis_name="s"),
    compiler_params=pltpu.CompilerParams(
        needs_layout_passes=False, use_tc_tiling_on_sc=False),
    scratch_types=[
        pltpu.VMEM_SHARED((N_NODES, D), jnp.bfloat16),
        pltpu.VMEM((EPW,), jnp.int32),
        pltpu.VMEM((EPW,), jnp.int32),
        pltpu.VMEM((2, CHUNK, D), jnp.bfloat16),
        pltpu.VMEM((2, CHUNK, D), jnp.bfloat16),
        pltpu.VMEM((EPW,), jnp.float32),
        pltpu.SemaphoreType.DMA,
    ],
)(_decoder_body)


@jax.jit
def kernel(z, edge_index):
    ei = edge_index.astype(jnp.int32)
    return _decoder(z.astype(jnp.bfloat16), ei[0], ei[1])


# speedup vs baseline: 2.2283x; 2.2283x over previous
import functools

import jax
import jax.numpy as jnp
from jax import lax
from jax.experimental import pallas as pl
from jax.experimental.pallas import tpu as pltpu
from jax.experimental.pallas import tpu_sc as plsc

N_NODES = 10000
D = 128
E = 320000
L = 16
NUM_CORES = 2
NUM_SUBCORES = 16
N_WORKERS = NUM_CORES * NUM_SUBCORES
EPW = E // N_WORKERS
CHUNK = 80
N_CHUNKS = EPW // CHUNK
UNROLL = 4


def _decoder_body(z_hbm, src_hbm, dst_hbm, out_hbm,
                  z_sh, idx_s_v, idx_d_v, rows_s, rows_d, out_v, sem):
    sid = lax.axis_index("s")
    wid = sid * NUM_CORES + lax.axis_index("c")
    base = wid * EPW

    @pl.when(sid == 0)
    def _():
        pltpu.sync_copy(z_hbm, z_sh)

    pltpu.sync_copy(src_hbm.at[pl.ds(base, EPW)], idx_s_v)
    pltpu.sync_copy(dst_hbm.at[pl.ds(base, EPW)], idx_d_v)
    plsc.subcore_barrier()

    lane = lax.iota(jnp.int32, L)
    rot = [jnp.bitwise_and(lane + (1 << r), L - 1) for r in range(4)]

    dnums = lax.GatherDimensionNumbers(
        offset_dims=(), collapsed_slice_dims=(0,), start_index_map=(0,))

    def _perm(v, idx):
        return lax.gather(v, idx[:, None], dnums, (1,),
                          mode=lax.GatherScatterMode.PROMISE_IN_BOUNDS)

    def _sum16(v):
        for r in range(4):
            v = v + _perm(v, rot[r])
        return v

    def _copies(ci, buf):
        off = ci * CHUNK
        return (
            pltpu.make_async_copy(
                z_sh.at[idx_s_v.at[pl.ds(off, CHUNK)]], rows_s.at[buf], sem),
            pltpu.make_async_copy(
                z_sh.at[idx_d_v.at[pl.ds(off, CHUNK)]], rows_d.at[buf], sem),
        )

    for c in _copies(0, 0):
        c.start()

    def chunk_body(ci, carry):
        off = ci * CHUNK
        buf = lax.rem(ci, 2)
        ci_next = jnp.minimum(ci + 1, N_CHUNKS - 1)
        for c in _copies(ci_next, 1 - buf):
            c.start()
        for c in _copies(ci, buf):
            c.wait()

        def group_body(g, carry2):
            res = jnp.zeros((L,), jnp.float32)
            for u in range(L):
                e = g * L + u
                acc0 = None
                acc1 = None
                for k in range(D // (2 * L)):
                    sv = rows_s[buf, e, pl.ds(k * 2 * L, 2 * L)]
                    dv = rows_d[buf, e, pl.ds(k * 2 * L, 2 * L)]
                    w = plsc.bitcast(sv * dv, jnp.uint32)
                    p0 = plsc.bitcast(w << jnp.uint32(16), jnp.float32)
                    p1 = plsc.bitcast(w & jnp.uint32(0xFFFF0000), jnp.float32)
                    acc0 = p0 if acc0 is None else acc0 + p0
                    acc1 = p1 if acc1 is None else acc1 + p1
                res = jnp.where(lane == u, _sum16(acc0 + acc1), res)
            out_v[pl.ds(off + g * L, L)] = res
            return carry2

        lax.fori_loop(0, CHUNK // L, group_body, 0)
        return carry

    lax.fori_loop(0, N_CHUNKS, chunk_body, 0)

    for c in _copies(N_CHUNKS - 1, N_CHUNKS % 2):
        c.wait()

    def sig_body(i, carry):
        x = out_v[pl.ds(i * L, L)]
        out_v[pl.ds(i * L, L)] = 1.0 / (1.0 + jnp.exp(-x))
        return carry

    lax.fori_loop(0, EPW // L, sig_body, 0)

    pltpu.sync_copy(out_v, out_hbm.at[pl.ds(base, EPW)])


_decoder = functools.partial(
    pl.kernel,
    out_type=jax.ShapeDtypeStruct((E,), jnp.float32),
    mesh=plsc.VectorSubcoreMesh(core_axis_name="c", subcore_axis_name="s"),
    compiler_params=pltpu.CompilerParams(
        needs_layout_passes=False, use_tc_tiling_on_sc=False),
    scratch_types=[
        pltpu.VMEM_SHARED((N_NODES, D), jnp.bfloat16),
        pltpu.VMEM((EPW,), jnp.int32),
        pltpu.VMEM((EPW,), jnp.int32),
        pltpu.VMEM((2, CHUNK, D), jnp.bfloat16),
        pltpu.VMEM((2, CHUNK, D), jnp.bfloat16),
        pltpu.VMEM((EPW,), jnp.float32),
        pltpu.SemaphoreType.DMA,
    ],
)(_decoder_body)


@jax.jit
def kernel(z, edge_index):
    ei = edge_index.astype(jnp.int32)
    return _decoder(z.astype(jnp.bfloat16), ei[0], ei[1])


# speedup vs baseline: 2.2348x; 1.0029x over previous
import functools

import jax
import jax.numpy as jnp
from jax import lax
from jax.experimental import pallas as pl
from jax.experimental.pallas import tpu as pltpu
from jax.experimental.pallas import tpu_sc as plsc

N_NODES = 10000
D = 128
E = 320000
L = 16
NUM_CORES = 2
NUM_SUBCORES = 16
N_WORKERS = NUM_CORES * NUM_SUBCORES
EPW = E // N_WORKERS
CHUNK = 80
N_CHUNKS = EPW // CHUNK
UNROLL = 4


def _decoder_body(z_hbm, src_hbm, dst_hbm, out_hbm,
                  z_sh, idx_s_v, idx_d_v, rows_s, rows_d, out_v, sem):
    sid = lax.axis_index("s")
    wid = sid * NUM_CORES + lax.axis_index("c")
    base = wid * EPW

    @pl.when(sid == 0)
    def _():
        pltpu.sync_copy(z_hbm, z_sh)

    pltpu.sync_copy(src_hbm.at[pl.ds(base, EPW)], idx_s_v)
    pltpu.sync_copy(dst_hbm.at[pl.ds(base, EPW)], idx_d_v)
    plsc.subcore_barrier()

    lane = lax.iota(jnp.int32, L)
    rot = [jnp.bitwise_and(lane + (1 << r), L - 1) for r in range(4)]

    dnums = lax.GatherDimensionNumbers(
        offset_dims=(), collapsed_slice_dims=(0,), start_index_map=(0,))

    def _perm(v, idx):
        return lax.gather(v, idx[:, None], dnums, (1,),
                          mode=lax.GatherScatterMode.PROMISE_IN_BOUNDS)

    def _sum16(v):
        for r in range(4):
            v = v + _perm(v, rot[r])
        return v

    def _copies(ci, buf):
        off = ci * CHUNK
        return (
            pltpu.make_async_copy(
                z_sh.at[idx_s_v.at[pl.ds(off, CHUNK)]], rows_s.at[buf], sem),
            pltpu.make_async_copy(
                z_sh.at[idx_d_v.at[pl.ds(off, CHUNK)]], rows_d.at[buf], sem),
        )

    for c in _copies(0, 0):
        c.start()

    def chunk_body(ci, carry):
        off = ci * CHUNK
        buf = lax.rem(ci, 2)
        ci_next = jnp.minimum(ci + 1, N_CHUNKS - 1)
        for c in _copies(ci_next, 1 - buf):
            c.start()
        for c in _copies(ci, buf):
            c.wait()

        def group_body(g, carry2):
            res = jnp.zeros((L,), jnp.float32)
            for u in range(L):
                e = g * L + u
                acc0 = None
                acc1 = None
                for k in range(D // (2 * L)):
                    sv = rows_s[buf, e, pl.ds(k * 2 * L, 2 * L)]
                    dv = rows_d[buf, e, pl.ds(k * 2 * L, 2 * L)]
                    w = plsc.bitcast(sv * dv, jnp.uint32)
                    p0 = plsc.bitcast(w << jnp.uint32(16), jnp.float32)
                    p1 = plsc.bitcast(w & jnp.uint32(0xFFFF0000), jnp.float32)
                    acc0 = p0 if acc0 is None else acc0 + p0
                    acc1 = p1 if acc1 is None else acc1 + p1
                res = jnp.where(lane == u, _sum16(acc0 + acc1), res)
            out_v[pl.ds(off + g * L, L)] = res
            return carry2

        lax.fori_loop(0, CHUNK // L, group_body, 0, unroll=CHUNK // L)
        return carry

    lax.fori_loop(0, N_CHUNKS, chunk_body, 0)

    for c in _copies(N_CHUNKS - 1, N_CHUNKS % 2):
        c.wait()

    def sig_body(i, carry):
        x = out_v[pl.ds(i * L, L)]
        out_v[pl.ds(i * L, L)] = 1.0 / (1.0 + jnp.exp(-x))
        return carry

    lax.fori_loop(0, EPW // L, sig_body, 0)

    pltpu.sync_copy(out_v, out_hbm.at[pl.ds(base, EPW)])


_decoder = functools.partial(
    pl.kernel,
    out_type=jax.ShapeDtypeStruct((E,), jnp.float32),
    mesh=plsc.VectorSubcoreMesh(core_axis_name="c", subcore_axis_name="s"),
    compiler_params=pltpu.CompilerParams(
        needs_layout_passes=False, use_tc_tiling_on_sc=False),
    scratch_types=[
        pltpu.VMEM_SHARED((N_NODES, D), jnp.bfloat16),
        pltpu.VMEM((EPW,), jnp.int32),
        pltpu.VMEM((EPW,), jnp.int32),
        pltpu.VMEM((2, CHUNK, D), jnp.bfloat16),
        pltpu.VMEM((2, CHUNK, D), jnp.bfloat16),
        pltpu.VMEM((EPW,), jnp.float32),
        pltpu.SemaphoreType.DMA,
    ],
)(_decoder_body)


@jax.jit
def kernel(z, edge_index):
    ei = edge_index.astype(jnp.int32)
    return _decoder(z.astype(jnp.bfloat16), ei[0], ei[1])
